# baseline (speedup 1.0000x reference)
import functools

import jax
import jax.numpy as jnp
from jax import lax
from jax.experimental import pallas as pl
from jax.experimental.pallas import tpu as pltpu
from jax.experimental.pallas import tpu_sc as plsc

_ENT_NUM = 1000000
_REL_NUM = 100
_DIM = 64
_BATCH = 16384

_NC = 2
_NS = 16
_NW = _NC * _NS
_BPW = _BATCH // _NW
_L = 16


def _transe_body(head_hbm, rel_hbm, tail_hbm, ent_hbm, relemb_hbm, out_hbm,
                 hidx_v, ridx_v, tidx_v, h_v, r_v, t_v, out_v, mat_v, sem):
    wid = lax.axis_index("s") * _NC + lax.axis_index("c")
    base = wid * _BPW

    pltpu.sync_copy(head_hbm.at[pl.ds(base, _BPW)], hidx_v)
    pltpu.sync_copy(rel_hbm.at[pl.ds(base, _BPW)], ridx_v)
    pltpu.sync_copy(tail_hbm.at[pl.ds(base, _BPW)], tidx_v)

    cp_h = pltpu.async_copy(ent_hbm.at[hidx_v], h_v, sem)
    cp_r = pltpu.async_copy(relemb_hbm.at[ridx_v], r_v, sem)
    cp_t = pltpu.async_copy(ent_hbm.at[tidx_v], t_v, sem)
    cp_h.wait()
    cp_r.wait()
    cp_t.wait()

    lanes = lax.iota(jnp.int32, _L)

    def block(blk, carry):
        i0 = blk * _L
        s = jnp.zeros((_L,), jnp.float32)
        for rr in range(_L):
            i = i0 + rr
            acc = None
            for j in range(_DIM // _L):
                h = h_v[i, pl.ds(j * _L, _L)]
                r = r_v[i, pl.ds(j * _L, _L)]
                t = t_v[i, pl.ds(j * _L, _L)]
                term = jnp.abs(h + r - t)
                acc = term if acc is None else acc + term
            s = jnp.where(lanes == rr, jnp.sum(acc), s)
        out_v[pl.ds(i0, _L)] = s
        return carry

    lax.fori_loop(0, _BPW // _L, block, 0)

    pltpu.sync_copy(out_v, out_hbm.at[pl.ds(base, _BPW)])


@jax.jit
def _transe(head, rel, tail, ent_embedding, rel_embedding):
    mesh = plsc.VectorSubcoreMesh(core_axis_name="c", subcore_axis_name="s")
    kern = pl.kernel(
        _transe_body,
        mesh=mesh,
        out_type=jax.ShapeDtypeStruct((_BATCH,), jnp.float32),
        scratch_types=[
            pltpu.VMEM((_BPW,), jnp.int32),
            pltpu.VMEM((_BPW,), jnp.int32),
            pltpu.VMEM((_BPW,), jnp.int32),
            pltpu.VMEM((_BPW, _DIM), jnp.float32),
            pltpu.VMEM((_BPW, _DIM), jnp.float32),
            pltpu.VMEM((_BPW, _DIM), jnp.float32),
            pltpu.VMEM((_BPW,), jnp.float32),
            pltpu.VMEM((_L, _L), jnp.float32),
            pltpu.SemaphoreType.DMA,
        ],
        compiler_params=pltpu.CompilerParams(
            needs_layout_passes=False, use_tc_tiling_on_sc=False),
    )
    return kern(head, rel, tail, ent_embedding, rel_embedding)


def kernel(head, rel, tail, ent_embedding, rel_embedding):
    return _transe(head, rel, tail, ent_embedding, rel_embedding)
